# baseline (speedup 1.0000x reference)
import jax
import jax.numpy as jnp
import numpy as np
from jax.experimental import pallas as pl


def _grid_sample(img, coords):
    B, C, H, W = img.shape
    x = ((coords[..., 0] + 1.0) * W - 1.0) / 2.0
    y = ((coords[..., 1] + 1.0) * H - 1.0) / 2.0
    x0 = jnp.floor(x); y0 = jnp.floor(y)
    x1 = x0 + 1.0; y1 = y0 + 1.0
    wa = (x1 - x) * (y1 - y)
    wb = (x1 - x) * (y - y0)
    wc = (x - x0) * (y1 - y)
    wd = (x - x0) * (y - y0)
    flat = img.reshape(B, C, H * W)
    N = coords.shape[1]

    def gather(ix, iy):
        valid = ((ix >= 0) & (ix <= W - 1) & (iy >= 0) & (iy <= H - 1)).astype(img.dtype)
        ixc = jnp.clip(ix, 0, W - 1).astype(jnp.int32)
        iyc = jnp.clip(iy, 0, H - 1).astype(jnp.int32)
        idx = iyc * W + ixc
        vals = jnp.take_along_axis(flat, jnp.broadcast_to(idx[:, None, :], (B, C, N)), axis=2)
        return vals * valid[:, None, :]

    out = (gather(x0, y0) * wa[:, None, :] + gather(x0, y1) * wb[:, None, :]
           + gather(x1, y0) * wc[:, None, :] + gather(x1, y1) * wd[:, None, :])
    return out


def _point_sample(img, points):
    return _grid_sample(img, 2.0 * points - 1.0)


def _sampling_points(mask, N, k, beta, key):
    B = mask.shape[0]
    mask_sorted = -jnp.sort(-mask, axis=1)
    k1, k2 = jax.random.split(key)
    over = jax.random.uniform(k1, (B, k * N, 2), dtype=mask.dtype)
    over_map = _point_sample(mask_sorted, over)
    unc = -1.0 * (over_map[:, 0] - over_map[:, 1])
    nb = int(beta * N)
    _, idx = jax.lax.top_k(unc, nb)
    importance = jnp.take_along_axis(over, idx[:, :, None], axis=1)
    coverage = jax.random.uniform(k2, (B, N - nb, 2), dtype=mask.dtype)
    return jnp.concatenate([importance, coverage], axis=1)


def _mlp_body(feat_ref, w1_ref, b1_ref, w2_ref, b2_ref, out_ref):
    feat = feat_ref[0]
    h = jnp.maximum(jnp.dot(feat, w1_ref[...], preferred_element_type=jnp.float32)
                    + b1_ref[...][None, :], 0.0)
    out_ref[0] = jnp.dot(h, w2_ref[...], preferred_element_type=jnp.float32) + b2_ref[...][None, :]


def kernel(fine, coarse, W1, b1, W2, b2):
    N, k, beta = 1024, 7, 0.75
    pkey = jax.random.key(42)
    points = _sampling_points(coarse, N, k, beta, pkey)
    coarse_s = _point_sample(coarse, points)
    fine_s = _point_sample(fine, points)
    feat = jnp.concatenate([coarse_s, fine_s], axis=1)
    featT = feat.transpose(0, 2, 1)
    B = featT.shape[0]
    W2p = jnp.pad(W2, ((0, 0), (0, 121)))
    b2p = jnp.pad(b2, (0, 121))
    out = pl.pallas_call(
        _mlp_body,
        grid=(B,),
        in_specs=[
            pl.BlockSpec((1, N, 135), lambda b: (b, 0, 0)),
            pl.BlockSpec((135, 256), lambda b: (0, 0)),
            pl.BlockSpec((256,), lambda b: (0,)),
            pl.BlockSpec((256, 128), lambda b: (0, 0)),
            pl.BlockSpec((128,), lambda b: (0,)),
        ],
        out_specs=pl.BlockSpec((1, N, 128), lambda b: (b, 0, 0)),
        out_shape=jax.ShapeDtypeStruct((B, N, 128), jnp.float32),
    )(featT, W1, b1, W2p, b2p)
    rend = out[:, :, :7].transpose(0, 2, 1)
    return rend, points


# speedup vs baseline: 1.3862x; 1.3862x over previous
import functools

import jax
import jax.numpy as jnp
import numpy as np
from jax import lax
from jax.experimental import pallas as pl
from jax.experimental.pallas import tpu as pltpu
from jax.experimental.pallas import tpu_sc as plsc

_B = 4
_N = 1024
_K = 7
_NB = 768
_NCOV = _N - _NB
_KN = _K * _N
_NTAB = _KN + _NCOV
_SC = 128
_SF = 256
_PC = _SC * _SC
_PF = _SF * _SF
_WPB = 8
_UCH = _KN // _WPB
_GCH = _N // _WPB
_SORTN = 8192


def _corner_tables(p, S):
    coords = 2.0 * p - 1.0
    x = ((coords[..., 0] + 1.0) * S - 1.0) / 2.0
    y = ((coords[..., 1] + 1.0) * S - 1.0) / 2.0
    x0 = jnp.floor(x)
    y0 = jnp.floor(y)
    x1 = x0 + 1.0
    y1 = y0 + 1.0
    ws = jnp.stack([(x1 - x) * (y1 - y), (x1 - x) * (y - y0),
                    (x - x0) * (y1 - y), (x - x0) * (y - y0)], axis=-1)
    xs = jnp.stack([x0, x0, x1, x1], axis=-1)
    ys = jnp.stack([y0, y1, y0, y1], axis=-1)
    valid = ((xs >= 0) & (xs <= S - 1) & (ys >= 0) & (ys <= S - 1)).astype(p.dtype)
    xc = jnp.clip(xs, 0, S - 1).astype(jnp.int32)
    yc = jnp.clip(ys, 0, S - 1).astype(jnp.int32)
    return yc * S + xc, ws * valid


def _consts():
    try:
        with jax.ensure_compile_time_eval():
            return jax.tree.map(np.asarray, _consts_impl())
    except Exception:
        return _consts_impl()


def _consts_impl():
    key = jax.random.key(42)
    k1, k2 = jax.random.split(key)
    over = jax.random.uniform(k1, (_B, _KN, 2), dtype=jnp.float32)
    cov = jax.random.uniform(k2, (_B, _NCOV, 2), dtype=jnp.float32)
    allp = jnp.concatenate([over, cov], axis=1)
    idx_c, w_c = _corner_tables(allp, _SC)
    idx_f, w_f = _corner_tables(allp, _SF)
    cidx = jnp.concatenate([idx_c, idx_f], axis=-1)
    cw = jnp.concatenate([w_c, w_f], axis=-1)
    uidx = jnp.transpose(idx_c[:, :_KN], (0, 2, 1))
    uw = jnp.transpose(w_c[:, :_KN], (0, 2, 1))
    ptab = jnp.concatenate(
        [cidx, lax.bitcast_convert_type(cw, jnp.int32),
         lax.bitcast_convert_type(allp, jnp.int32),
         jnp.zeros((_B, _NTAB, 110), jnp.int32)], axis=-1)
    return ptab, uidx, uw


def _gap_body(c_ref, g_ref):
    x = c_ref[0]
    m1 = jnp.full((1, _PC), -jnp.inf, jnp.float32)
    m2 = jnp.full((1, _PC), -jnp.inf, jnp.float32)
    for c in range(_K):
        v = x[c][None, :]
        gt = v > m1
        m2 = jnp.where(gt, m1, jnp.maximum(m2, v))
        m1 = jnp.maximum(m1, v)
    g_ref[0, 0] = m1[0]
    g_ref[0, 1] = m2[0]


def _coarset_body(c_ref, ct_ref):
    xt = jnp.swapaxes(c_ref[0], 0, 1)
    ct_ref[0] = jnp.concatenate([xt, jnp.zeros((1024, 121), jnp.float32)],
                                axis=1)


def _finet_body(f_ref, ft_ref):
    ft_ref[0] = jnp.swapaxes(f_ref[0], 0, 1)


def _unc_body(g_hbm, uidx_hbm, uw_hbm, unc_hbm, m1v, m2v, idxv, wv, uncv, sem):
    nc = 2
    wid = lax.axis_index("s") * nc + lax.axis_index("c")
    b = wid // _WPB
    j = wid % _WPB
    base = j * _UCH
    pltpu.sync_copy(g_hbm.at[b, 0], m1v)
    pltpu.sync_copy(g_hbm.at[b, 1], m2v)
    pltpu.sync_copy(uidx_hbm.at[b, :, pl.ds(base, _UCH)], idxv)
    pltpu.sync_copy(uw_hbm.at[b, :, pl.ds(base, _UCH)], wv)

    def body(t, carry):
        off = pl.multiple_of(t * 16, 16)
        acc0 = jnp.zeros((16,), jnp.float32)
        acc1 = jnp.zeros((16,), jnp.float32)
        for c in range(4):
            iv = idxv[c, pl.ds(off, 16)]
            w = wv[c, pl.ds(off, 16)]
            acc0 = acc0 + plsc.load_gather(m1v, [iv]) * w
            acc1 = acc1 + plsc.load_gather(m2v, [iv]) * w
        uncv[pl.ds(off, 16)] = acc1 - acc0
        return carry

    lax.fori_loop(0, _UCH // 16, body, 0)
    pltpu.sync_copy(uncv, unc_hbm.at[b, pl.ds(base, _UCH)])


def _topk_body(unc_ref, sel_ref):
    keys = jnp.concatenate(
        [unc_ref[...], jnp.full((_B, _SORTN - _KN), -jnp.inf, jnp.float32)],
        axis=1)
    idx = lax.broadcasted_iota(jnp.int32, (_B, _SORTN), 1)
    pos = idx
    n = _SORTN
    for kk in range(1, 14):
        for jj in range(kk - 1, -1, -1):
            s = 1 << jj
            kl = jnp.concatenate([keys[:, s:], keys[:, :s]], axis=1)
            il = jnp.concatenate([idx[:, s:], idx[:, :s]], axis=1)
            kr = jnp.concatenate([keys[:, n - s:], keys[:, :n - s]], axis=1)
            ir = jnp.concatenate([idx[:, n - s:], idx[:, :n - s]], axis=1)
            low = (pos & s) == 0
            pk = jnp.where(low, kl, kr)
            pi = jnp.where(low, il, ir)
            desc = (pos & (1 << kk)) == 0
            beats_desc = (keys > pk) | ((keys == pk) & (idx < pi))
            beats_asc = (keys < pk) | ((keys == pk) & (idx > pi))
            beats = (desc & beats_desc) | (~desc & beats_asc)
            keep = ~(beats ^ low)
            keys = jnp.where(keep, keys, pk)
            idx = jnp.where(keep, idx, pi)
    sel_ref[:, :_NB] = idx[:, :_NB]
    sel_ref[:, _NB:] = lax.broadcasted_iota(jnp.int32, (_B, _NCOV), 1) + _KN


def _gather_body(sel_hbm, ptab_hbm, ct_hbm, ft_hbm,
                 pts_o, w_o, cg_o, fg_o,
                 selv, prow, clc, clf, wv, ptsv, rows4, sem):
    nc = 2
    wid = lax.axis_index("s") * nc + lax.axis_index("c")
    b = wid // _WPB
    j = wid % _WPB
    base = j * _GCH
    pltpu.sync_copy(sel_hbm.at[b, pl.ds(base, _GCH)], selv)
    pltpu.async_copy(ptab_hbm.at[b].at[selv], prow, sem).wait()
    for t in range(_GCH // 16):
        rows = lax.iota(jnp.int32, 16) + t * 16

        def col(cc):
            return plsc.load_gather(prow, [rows, jnp.full((16,), cc, jnp.int32)])

        for c in range(4):
            clc[c, pl.ds(t * 16, 16)] = col(c)
            clf[c, pl.ds(t * 16, 16)] = col(4 + c)
        for c in range(8):
            wv[c, pl.ds(t * 16, 16)] = plsc.bitcast(col(8 + c), jnp.float32)
        ptsv[0, pl.ds(t * 16, 16)] = plsc.bitcast(col(16), jnp.float32)
        ptsv[1, pl.ds(t * 16, 16)] = plsc.bitcast(col(17), jnp.float32)
    pltpu.sync_copy(ptsv, pts_o.at[b, :, pl.ds(base, _GCH)])
    pltpu.sync_copy(wv, w_o.at[b, :, pl.ds(base, _GCH)])
    for c in range(4):
        pltpu.async_copy(ft_hbm.at[b].at[clf.at[c]], rows4.at[c], sem).wait()
    pltpu.sync_copy(rows4, fg_o.at[b, :, pl.ds(base, _GCH), :])
    for c in range(4):
        pltpu.async_copy(ct_hbm.at[b].at[clc.at[c]], rows4.at[c], sem).wait()
    pltpu.sync_copy(rows4, cg_o.at[b, :, pl.ds(base, _GCH), :])


def _mlp_body(cg_ref, fg_ref, w_ref, w1_ref, b1_ref, w2_ref, b2_ref, out_ref):
    w = jnp.swapaxes(w_ref[0], 0, 1)
    cs = jnp.zeros((_N, 128), jnp.float32)
    fs = jnp.zeros((_N, 128), jnp.float32)
    for c in range(4):
        cs = cs + w[:, c:c + 1] * cg_ref[0, c]
        fs = fs + w[:, 4 + c:5 + c] * fg_ref[0, c]
    feat = jnp.concatenate([cs[:, :_K], fs], axis=1)
    h = jnp.dot(feat, w1_ref[...], preferred_element_type=jnp.float32)
    h = jnp.maximum(h + b1_ref[...][None, :], 0.0)
    out_ref[0] = (jnp.dot(h, w2_ref[...], preferred_element_type=jnp.float32)
                  + b2_ref[...][None, :])


def kernel(fine, coarse, W1, b1, W2, b2):
    ptab, uidx, uw = _consts()
    cflat = coarse.reshape(_B, _K, _PC)
    fflat = fine.reshape(_B, 128, _PF)

    g = pl.pallas_call(
        _gap_body,
        grid=(_B,),
        in_specs=[pl.BlockSpec((1, _K, _PC), lambda b: (b, 0, 0))],
        out_specs=pl.BlockSpec((1, 2, _PC), lambda b: (b, 0, 0)),
        out_shape=jax.ShapeDtypeStruct((_B, 2, _PC), jnp.float32),
    )(cflat)

    coarset = pl.pallas_call(
        _coarset_body,
        grid=(_B, _PC // 1024),
        in_specs=[
            pl.BlockSpec((1, _K, 1024), lambda b, j: (b, 0, j)),
        ],
        out_specs=pl.BlockSpec((1, 1024, 128), lambda b, j: (b, j, 0)),
        out_shape=jax.ShapeDtypeStruct((_B, _PC, 128), jnp.float32),
    )(cflat)

    finet = pl.pallas_call(
        _finet_body,
        grid=(_B, _PF // 1024),
        in_specs=[pl.BlockSpec((1, 128, 1024), lambda b, j: (b, 0, j))],
        out_specs=pl.BlockSpec((1, 1024, 128), lambda b, j: (b, j, 0)),
        out_shape=jax.ShapeDtypeStruct((_B, _PF, 128), jnp.float32),
    )(fflat)

    mesh = plsc.VectorSubcoreMesh(core_axis_name="c", subcore_axis_name="s")

    unc = pl.kernel(
        _unc_body,
        out_type=jax.ShapeDtypeStruct((_B, _KN), jnp.float32),
        mesh=mesh,
        compiler_params=pltpu.CompilerParams(needs_layout_passes=False),
        scratch_types=[
            pltpu.VMEM((_PC,), jnp.float32),
            pltpu.VMEM((_PC,), jnp.float32),
            pltpu.VMEM((4, _UCH), jnp.int32),
            pltpu.VMEM((4, _UCH), jnp.float32),
            pltpu.VMEM((_UCH,), jnp.float32),
            pltpu.SemaphoreType.DMA,
        ],
    )(g, jnp.asarray(uidx), jnp.asarray(uw))

    sel = pl.pallas_call(
        _topk_body,
        in_specs=[pl.BlockSpec((_B, _KN), lambda: (0, 0))],
        out_specs=pl.BlockSpec((_B, _N), lambda: (0, 0)),
        out_shape=jax.ShapeDtypeStruct((_B, _N), jnp.int32),
    )(unc)

    pts, wsel, cg, fg = pl.kernel(
        _gather_body,
        out_type=[
            jax.ShapeDtypeStruct((_B, 2, _N), jnp.float32),
            jax.ShapeDtypeStruct((_B, 8, _N), jnp.float32),
            jax.ShapeDtypeStruct((_B, 4, _N, 128), jnp.float32),
            jax.ShapeDtypeStruct((_B, 4, _N, 128), jnp.float32),
        ],
        mesh=mesh,
        compiler_params=pltpu.CompilerParams(needs_layout_passes=False),
        scratch_types=[
            pltpu.VMEM((_GCH,), jnp.int32),
            pltpu.VMEM((_GCH, 128), jnp.int32),
            pltpu.VMEM((4, _GCH), jnp.int32),
            pltpu.VMEM((4, _GCH), jnp.int32),
            pltpu.VMEM((8, _GCH), jnp.float32),
            pltpu.VMEM((2, _GCH), jnp.float32),
            pltpu.VMEM((4, _GCH, 128), jnp.float32),
            pltpu.SemaphoreType.DMA,
        ],
    )(sel, jnp.asarray(ptab), coarset, finet)

    w2p = jnp.pad(W2, ((0, 0), (0, 128 - _K)))
    b2p = jnp.pad(b2, (0, 128 - _K))
    out = pl.pallas_call(
        _mlp_body,
        grid=(_B,),
        in_specs=[
            pl.BlockSpec((1, 4, _N, 128), lambda b: (b, 0, 0, 0)),
            pl.BlockSpec((1, 4, _N, 128), lambda b: (b, 0, 0, 0)),
            pl.BlockSpec((1, 8, _N), lambda b: (b, 0, 0)),
            pl.BlockSpec((135, 256), lambda b: (0, 0)),
            pl.BlockSpec((256,), lambda b: (0,)),
            pl.BlockSpec((256, 128), lambda b: (0, 0)),
            pl.BlockSpec((128,), lambda b: (0,)),
        ],
        out_specs=pl.BlockSpec((1, _N, 128), lambda b: (b, 0, 0)),
        out_shape=jax.ShapeDtypeStruct((_B, _N, 128), jnp.float32),
    )(cg, fg, wsel, W1, b1, w2p, b2p)

    rend = out[:, :, :_K].transpose(0, 2, 1)
    return rend, pts.transpose(0, 2, 1)
